# baseline (speedup 1.0000x reference)
import jax
import jax.numpy as jnp
from jax import lax
from jax.experimental import pallas as pl
from jax.experimental.pallas import tpu as pltpu
from jax.experimental.pallas import tpu_sc as plsc

D = 128
NEIGH = 32
NC = 2
NS = 16
NWT = 14
NW = NC * NWT
BPW = 368
B_PAD = NW * BPW
C = 4
E = C * NEIGH
NCHUNK = BPW // C
NBUF = 4
NQUAD = NCHUNK // NBUF
CN = 184


def _tc1_body(x_ref, w_ref, b_ref, o_ref):
    acc = jnp.dot(x_ref[...], w_ref[...], preferred_element_type=jnp.float32)
    o_ref[...] = jnp.maximum(acc + b_ref[...], 0.0)


def _transform_table(features, W_dense, b_dense):
    n = features.shape[0]
    blk = 1000
    grid = n // blk
    return pl.pallas_call(
        _tc1_body,
        grid=(grid,),
        in_specs=[
            pl.BlockSpec((blk, D), lambda i: (i, 0)),
            pl.BlockSpec((D, D), lambda i: (0, 0)),
            pl.BlockSpec((1, D), lambda i: (0, 0)),
        ],
        out_specs=pl.BlockSpec((blk, D), lambda i: (i, 0)),
        out_shape=jax.ShapeDtypeStruct((n, D), jnp.float32),
    )(features, W_dense, b_dense.reshape(1, D))


def _sc_body(t_hbm, feat_hbm, neigh_hbm, node_hbm, pooled_hbm, nodef_hbm,
             idxall_v, b0, b1, b2, b3, pooled_v, nidx_v,
             s0, s1, s2, s3):
    bufs = (b0, b1, b2, b3)
    sems = (s0, s1, s2, s3)
    cid = lax.axis_index("c")
    sid = lax.axis_index("s")

    @pl.when(sid < NWT)
    def _():
        wid = cid * NWT + sid
        base = wid * BPW

        def node_chunk(j, carry):
            off = base + j * CN
            pltpu.sync_copy(node_hbm.at[pl.ds(off, CN)], nidx_v)
            pltpu.async_copy(feat_hbm.at[nidx_v],
                             pooled_v.at[pl.ds(0, CN)], s0).wait()
            pltpu.sync_copy(pooled_v.at[pl.ds(0, CN)],
                            nodef_hbm.at[pl.ds(off, CN)])
            return carry
        with jax.named_scope("node_gather"):
            lax.fori_loop(0, BPW // CN, node_chunk, 0)

        with jax.named_scope("idx_prefetch"):
            pltpu.sync_copy(neigh_hbm.at[pl.ds(base * NEIGH, BPW * NEIGH)],
                            idxall_v)

        def gather_start(g, rows_v, sem):
            pltpu.async_copy(t_hbm.at[idxall_v.at[pl.ds(g * E, E)]],
                             rows_v, sem)

        def gather_wait(rows_v, sem):
            pltpu.make_async_copy(t_hbm.at[idxall_v.at[pl.ds(0, E)]],
                                  rows_v, sem).wait()

        def compute(g, rows_v):
            def nbody(n, accs):
                new = []
                for c in range(C):
                    for d in range(D // 16):
                        new.append(accs[c * (D // 16) + d]
                                   + rows_v[c * NEIGH + n, pl.ds(d * 16, 16)])
                return tuple(new)
            init = tuple(jnp.zeros((16,), jnp.float32)
                         for _ in range(C * (D // 16)))
            accs = lax.fori_loop(0, NEIGH, nbody, init)
            for c in range(C):
                for d in range(D // 16):
                    pooled_v[g * C + c, pl.ds(d * 16, 16)] = (
                        accs[c * (D // 16) + d] * (1.0 / NEIGH))

        with jax.named_scope("pool_loop"):
            for b in range(NBUF - 1):
                gather_start(b, bufs[b], sems[b])

            def quad(i, carry):
                a = NBUF * i
                for b in range(NBUF):
                    g = a + b

                    @pl.when(g + NBUF - 1 < NCHUNK)
                    def _():
                        gather_start(g + NBUF - 1, bufs[(b + NBUF - 1) % NBUF],
                                     sems[(b + NBUF - 1) % NBUF])
                    gather_wait(bufs[b], sems[b])
                    compute(g, bufs[b])
                return carry
            lax.fori_loop(0, NQUAD, quad, 0)
        with jax.named_scope("pooled_writeout"):
            pltpu.sync_copy(pooled_v, pooled_hbm.at[pl.ds(base, BPW)])


def _sc_gather_pool(T, features, neigh_flat, node_flat):
    mesh = plsc.VectorSubcoreMesh(core_axis_name="c", subcore_axis_name="s")
    return pl.kernel(
        _sc_body,
        out_type=(
            jax.ShapeDtypeStruct((B_PAD, D), jnp.float32),
            jax.ShapeDtypeStruct((B_PAD, D), jnp.float32),
        ),
        mesh=mesh,
        scratch_types=[
            pltpu.VMEM((BPW * NEIGH,), jnp.int32),
            pltpu.VMEM((E, D), jnp.float32),
            pltpu.VMEM((E, D), jnp.float32),
            pltpu.VMEM((E, D), jnp.float32),
            pltpu.VMEM((E, D), jnp.float32),
            pltpu.VMEM((BPW, D), jnp.float32),
            pltpu.VMEM((CN,), jnp.int32),
            pltpu.SemaphoreType.DMA,
            pltpu.SemaphoreType.DMA,
            pltpu.SemaphoreType.DMA,
            pltpu.SemaphoreType.DMA,
        ],
    )(T, features, neigh_flat, node_flat)


def _tc2_body(nf_ref, pv_ref, w1_ref, w2_ref, o_ref):
    acc = jnp.dot(nf_ref[...], w1_ref[...], preferred_element_type=jnp.float32)
    acc = acc + jnp.dot(pv_ref[...], w2_ref[...], preferred_element_type=jnp.float32)
    o_ref[...] = jnp.maximum(acc, 0.0)


def _final_matmul(nodef, pooled, w1, w2, b):
    blk = 1000
    grid = b // blk
    return pl.pallas_call(
        _tc2_body,
        grid=(grid,),
        in_specs=[
            pl.BlockSpec((blk, D), lambda i: (i, 0)),
            pl.BlockSpec((blk, D), lambda i: (i, 0)),
            pl.BlockSpec((D, D), lambda i: (0, 0)),
            pl.BlockSpec((D, D), lambda i: (0, 0)),
        ],
        out_specs=pl.BlockSpec((blk, D), lambda i: (i, 0)),
        out_shape=jax.ShapeDtypeStruct((b, D), jnp.float32),
    )(nodef, pooled, w1, w2)


def kernel(features, node, neighbours, W_dense, b_dense, neigh_weights):
    b = node.shape[0]
    pad = B_PAD - b
    node_flat = node.reshape(b).astype(jnp.int32)
    node_p = jnp.pad(node_flat, (0, pad))
    neigh_p = jnp.pad(neighbours.astype(jnp.int32), ((0, pad), (0, 0)))
    neigh_flat = neigh_p.reshape(B_PAD * NEIGH)

    T = _transform_table(features, W_dense, b_dense)
    pooled, nodef = _sc_gather_pool(T, features, neigh_flat, node_p)
    out = _final_matmul(nodef, pooled, neigh_weights[:D], neigh_weights[D:], b)
    return out


# speedup vs baseline: 1.2538x; 1.2538x over previous
import jax
import jax.numpy as jnp
from jax import lax
from jax.experimental import pallas as pl
from jax.experimental.pallas import tpu as pltpu
from jax.experimental.pallas import tpu_sc as plsc

D = 128
NEIGH = 32
NC = 2
NS = 16
NFAST = 14
QF = 352
QS = 96
B_PAD = NC * (NFAST * QF + 2 * QS)
FAST_TOTAL = NC * NFAST * QF
C = 4
E = C * NEIGH
NBUF = 4
CNF = 176
CNS = 96


def _tc1_body(x_ref, w_ref, b_ref, o_ref):
    acc = jnp.dot(x_ref[...], w_ref[...], preferred_element_type=jnp.float32)
    o_ref[...] = jnp.maximum(acc + b_ref[...], 0.0)


def _transform_table(features, W_dense, b_dense):
    n = features.shape[0]
    blk = 1000
    grid = n // blk
    return pl.pallas_call(
        _tc1_body,
        grid=(grid,),
        in_specs=[
            pl.BlockSpec((blk, D), lambda i: (i, 0)),
            pl.BlockSpec((D, D), lambda i: (0, 0)),
            pl.BlockSpec((1, D), lambda i: (0, 0)),
        ],
        out_specs=pl.BlockSpec((blk, D), lambda i: (i, 0)),
        out_shape=jax.ShapeDtypeStruct((n, D), jnp.float32),
    )(features, W_dense, b_dense.reshape(1, D))


def _sc_body(t_hbm, feat_hbm, neigh_hbm, node_hbm, pooled_hbm, nodef_hbm,
             idxall_v, b0, b1, b2, b3, pooled_v, nidx_v,
             s0, s1, s2, s3):
    bufs = (b0, b1, b2, b3)
    sems = (s0, s1, s2, s3)
    cid = lax.axis_index("c")
    sid = lax.axis_index("s")

    def gather_start(g, rows_v, sem):
        pltpu.async_copy(t_hbm.at[idxall_v.at[pl.ds(g * E, E)]], rows_v, sem)

    def gather_wait(rows_v, sem):
        pltpu.make_async_copy(t_hbm.at[idxall_v.at[pl.ds(0, E)]],
                              rows_v, sem).wait()

    def compute(g, rows_v):
        def nbody(n, accs):
            new = []
            for c in range(C):
                for d in range(D // 16):
                    new.append(accs[c * (D // 16) + d]
                               + rows_v[c * NEIGH + n, pl.ds(d * 16, 16)])
            return tuple(new)
        init = tuple(jnp.zeros((16,), jnp.float32)
                     for _ in range(C * (D // 16)))
        accs = lax.fori_loop(0, NEIGH, nbody, init)
        for c in range(C):
            for d in range(D // 16):
                pooled_v[g * C + c, pl.ds(d * 16, 16)] = (
                    accs[c * (D // 16) + d] * (1.0 / NEIGH))

    def run(base, rows, cn, nnode):
        def node_chunk(j, carry):
            off = base + j * cn
            pltpu.sync_copy(node_hbm.at[pl.ds(off, cn)],
                            nidx_v.at[pl.ds(0, cn)])
            pltpu.async_copy(feat_hbm.at[nidx_v.at[pl.ds(0, cn)]],
                             pooled_v.at[pl.ds(0, cn)], s0).wait()
            pltpu.sync_copy(pooled_v.at[pl.ds(0, cn)],
                            nodef_hbm.at[pl.ds(off, cn)])
            return carry
        with jax.named_scope("node_gather"):
            lax.fori_loop(0, nnode, node_chunk, 0)

        with jax.named_scope("idx_prefetch"):
            pltpu.sync_copy(neigh_hbm.at[pl.ds(base * NEIGH, rows * NEIGH)],
                            idxall_v.at[pl.ds(0, rows * NEIGH)])

        nchunk = rows // C
        nquad = nchunk // NBUF
        with jax.named_scope("pool_loop"):
            for b in range(NBUF - 1):
                gather_start(b, bufs[b], sems[b])

            def quad(i, carry):
                a = NBUF * i
                for b in range(NBUF):
                    g = a + b

                    @pl.when(g + NBUF - 1 < nchunk)
                    def _():
                        gather_start(g + NBUF - 1,
                                     bufs[(b + NBUF - 1) % NBUF],
                                     sems[(b + NBUF - 1) % NBUF])
                    gather_wait(bufs[b], sems[b])
                    compute(g, bufs[b])
                return carry
            lax.fori_loop(0, nquad, quad, 0)
        with jax.named_scope("pooled_writeout"):
            pltpu.sync_copy(pooled_v.at[pl.ds(0, rows)],
                            pooled_hbm.at[pl.ds(base, rows)])

    @pl.when(sid < NFAST)
    def _():
        run((cid * NFAST + sid) * QF, QF, CNF, QF // CNF)

    @pl.when(sid >= NFAST)
    def _():
        run(FAST_TOTAL + (cid * 2 + (sid - NFAST)) * QS, QS, CNS, QS // CNS)


def _sc_gather_pool(T, features, neigh_flat, node_flat):
    mesh = plsc.VectorSubcoreMesh(core_axis_name="c", subcore_axis_name="s")
    return pl.kernel(
        _sc_body,
        out_type=(
            jax.ShapeDtypeStruct((B_PAD, D), jnp.float32),
            jax.ShapeDtypeStruct((B_PAD, D), jnp.float32),
        ),
        mesh=mesh,
        scratch_types=[
            pltpu.VMEM((QF * NEIGH,), jnp.int32),
            pltpu.VMEM((E, D), jnp.float32),
            pltpu.VMEM((E, D), jnp.float32),
            pltpu.VMEM((E, D), jnp.float32),
            pltpu.VMEM((E, D), jnp.float32),
            pltpu.VMEM((QF, D), jnp.float32),
            pltpu.VMEM((CNF,), jnp.int32),
            pltpu.SemaphoreType.DMA,
            pltpu.SemaphoreType.DMA,
            pltpu.SemaphoreType.DMA,
            pltpu.SemaphoreType.DMA,
        ],
    )(T, features, neigh_flat, node_flat)


def _tc2_body(nf_ref, pv_ref, w1_ref, w2_ref, o_ref):
    acc = jnp.dot(nf_ref[...], w1_ref[...], preferred_element_type=jnp.float32)
    acc = acc + jnp.dot(pv_ref[...], w2_ref[...], preferred_element_type=jnp.float32)
    o_ref[...] = jnp.maximum(acc, 0.0)


def _final_matmul(nodef, pooled, w1, w2, b):
    blk = 1000
    grid = b // blk
    return pl.pallas_call(
        _tc2_body,
        grid=(grid,),
        in_specs=[
            pl.BlockSpec((blk, D), lambda i: (i, 0)),
            pl.BlockSpec((blk, D), lambda i: (i, 0)),
            pl.BlockSpec((D, D), lambda i: (0, 0)),
            pl.BlockSpec((D, D), lambda i: (0, 0)),
        ],
        out_specs=pl.BlockSpec((blk, D), lambda i: (i, 0)),
        out_shape=jax.ShapeDtypeStruct((b, D), jnp.float32),
    )(nodef, pooled, w1, w2)


def kernel(features, node, neighbours, W_dense, b_dense, neigh_weights):
    b = node.shape[0]
    pad = B_PAD - b
    node_flat = node.reshape(b).astype(jnp.int32)
    node_p = jnp.pad(node_flat, (0, pad))
    neigh_p = jnp.pad(neighbours.astype(jnp.int32), ((0, pad), (0, 0)))
    neigh_flat = neigh_p.reshape(B_PAD * NEIGH)

    T = _transform_table(features, W_dense, b_dense)
    pooled, nodef = _sc_gather_pool(T, features, neigh_flat, node_p)
    out = _final_matmul(nodef, pooled, neigh_weights[:D], neigh_weights[D:], b)
    return out


# speedup vs baseline: 3.6133x; 2.8820x over previous
import jax
import jax.numpy as jnp
from jax import lax
from jax.experimental import pallas as pl
from jax.experimental.pallas import tpu as pltpu
from jax.experimental.pallas import tpu_sc as plsc

D = 128
NEIGH = 32
NC = 2
NS = 16
NFAST = 14
QF = 352
QS = 96
B_PAD = NC * (NFAST * QF + 2 * QS)
FAST_TOTAL = NC * NFAST * QF
C = 4
E = C * NEIGH
NBUF = 4
CNF = 176
CNS = 96


def _tc1_body(x_ref, w_ref, b_ref, o_ref):
    acc = jnp.dot(x_ref[...], w_ref[...], preferred_element_type=jnp.float32)
    o_ref[...] = jnp.maximum(acc + b_ref[...], 0.0)


def _transform_table(features, W_dense, b_dense):
    n = features.shape[0]
    blk = 1000
    grid = n // blk
    return pl.pallas_call(
        _tc1_body,
        grid=(grid,),
        in_specs=[
            pl.BlockSpec((blk, D), lambda i: (i, 0)),
            pl.BlockSpec((D, D), lambda i: (0, 0)),
            pl.BlockSpec((1, D), lambda i: (0, 0)),
        ],
        out_specs=pl.BlockSpec((blk, D), lambda i: (i, 0)),
        out_shape=jax.ShapeDtypeStruct((n, D), jnp.float32),
    )(features, W_dense, b_dense.reshape(1, D))


def _sc_body(t_hbm, feat_hbm, neigh_hbm, node_hbm, pooled_hbm, nodef_hbm,
             idxall_v, b0, b1, b2, b3, pooled_v, nidx_v,
             s0, s1, s2, s3):
    bufs = (b0, b1, b2, b3)
    sems = (s0, s1, s2, s3)
    cid = lax.axis_index("c")
    sid = lax.axis_index("s")

    def gather_start(g, rows_v, sem):
        pltpu.async_copy(t_hbm.at[idxall_v.at[pl.ds(g * E, E)]], rows_v, sem)

    def gather_wait(rows_v, sem):
        pltpu.make_async_copy(t_hbm.at[idxall_v.at[pl.ds(0, E)]],
                              rows_v, sem).wait()

    def compute(g, rows_v):
        def nbody(n, accs):
            new = []
            for c in range(C):
                for d in range(D // 16):
                    new.append(accs[c * (D // 16) + d]
                               + rows_v[c * NEIGH + n, pl.ds(d * 16, 16)])
            return tuple(new)
        init = tuple(jnp.zeros((16,), jnp.float32)
                     for _ in range(C * (D // 16)))
        accs = lax.fori_loop(0, NEIGH, nbody, init)
        for c in range(C):
            for d in range(D // 16):
                pooled_v[g * C + c, pl.ds(d * 16, 16)] = (
                    accs[c * (D // 16) + d] * (1.0 / NEIGH))

    def run(base, rows, cn, nnode):
        def node_chunk(j, carry):
            off = base + j * cn
            pltpu.sync_copy(node_hbm.at[pl.ds(off, cn)],
                            nidx_v.at[pl.ds(0, cn)])
            pltpu.async_copy(feat_hbm.at[nidx_v.at[pl.ds(0, cn)]],
                             pooled_v.at[pl.ds(0, cn)], s0).wait()
            pltpu.sync_copy(pooled_v.at[pl.ds(0, cn)],
                            nodef_hbm.at[pl.ds(off, cn)])
            return carry
        with jax.named_scope("node_gather"):
            lax.fori_loop(0, nnode, node_chunk, 0)

        with jax.named_scope("idx_prefetch"):
            pltpu.sync_copy(neigh_hbm.at[pl.ds(base * NEIGH, rows * NEIGH)],
                            idxall_v.at[pl.ds(0, rows * NEIGH)])

        nchunk = rows // C
        nquad = nchunk // NBUF
        with jax.named_scope("pool_loop"):
            for b in range(NBUF - 1):
                gather_start(b, bufs[b], sems[b])

            def quad(i, carry):
                a = NBUF * i
                for b in range(NBUF):
                    g = a + b

                    @pl.when(g + NBUF - 1 < nchunk)
                    def _():
                        gather_start(g + NBUF - 1,
                                     bufs[(b + NBUF - 1) % NBUF],
                                     sems[(b + NBUF - 1) % NBUF])
                    gather_wait(bufs[b], sems[b])
                    compute(g, bufs[b])
                return carry
            lax.fori_loop(0, nquad, quad, 0)
        with jax.named_scope("pooled_writeout"):
            pltpu.sync_copy(pooled_v.at[pl.ds(0, rows)],
                            pooled_hbm.at[pl.ds(base, rows)])

    @pl.when(sid < NFAST)
    def _():
        run((cid * NFAST + sid) * QF, QF, CNF, QF // CNF)

    @pl.when(sid >= NFAST)
    def _():
        run(FAST_TOTAL + (cid * 2 + (sid - NFAST)) * QS, QS, CNS, QS // CNS)


def _sc_gather_pool(T, features, neigh_flat, node_flat):
    mesh = plsc.VectorSubcoreMesh(core_axis_name="c", subcore_axis_name="s")
    return pl.kernel(
        _sc_body,
        out_type=(
            jax.ShapeDtypeStruct((B_PAD, D), jnp.float32),
            jax.ShapeDtypeStruct((B_PAD, D), jnp.float32),
        ),
        mesh=mesh,
        scratch_types=[
            pltpu.VMEM((QF * NEIGH,), jnp.int32),
            pltpu.VMEM((E, D), jnp.float32),
            pltpu.VMEM((E, D), jnp.float32),
            pltpu.VMEM((E, D), jnp.float32),
            pltpu.VMEM((E, D), jnp.float32),
            pltpu.VMEM((QF, D), jnp.float32),
            pltpu.VMEM((CNF,), jnp.int32),
            pltpu.SemaphoreType.DMA,
            pltpu.SemaphoreType.DMA,
            pltpu.SemaphoreType.DMA,
            pltpu.SemaphoreType.DMA,
        ],
    )(T, features, neigh_flat, node_flat)


def _tc2_body(nf_ref, pv_ref, w1_ref, w2_ref, o_ref):
    acc = jnp.dot(nf_ref[...], w1_ref[...], preferred_element_type=jnp.float32)
    acc = acc + jnp.dot(pv_ref[...], w2_ref[...], preferred_element_type=jnp.float32)
    o_ref[...] = jnp.maximum(acc, 0.0)


def _final_matmul(nodef, pooled, w1, w2, b):
    blk = 1000
    grid = b // blk
    return pl.pallas_call(
        _tc2_body,
        grid=(grid,),
        in_specs=[
            pl.BlockSpec((blk, D), lambda i: (i, 0)),
            pl.BlockSpec((blk, D), lambda i: (i, 0)),
            pl.BlockSpec((D, D), lambda i: (0, 0)),
            pl.BlockSpec((D, D), lambda i: (0, 0)),
        ],
        out_specs=pl.BlockSpec((blk, D), lambda i: (i, 0)),
        out_shape=jax.ShapeDtypeStruct((b, D), jnp.float32),
    )(nodef, pooled, w1, w2)


def kernel(features, node, neighbours, W_dense, b_dense, neigh_weights):
    b = node.shape[0]
    n_nodes = features.shape[0]
    pad = B_PAD - b
    node_flat = node.reshape(b).astype(jnp.int32)
    node_pad = (jnp.arange(pad, dtype=jnp.int32) * 1031) % n_nodes
    node_p = jnp.concatenate([node_flat, node_pad])
    neigh_pad = ((jnp.arange(pad * NEIGH, dtype=jnp.int32) * 1031) % n_nodes)
    neigh_flat = jnp.concatenate(
        [neighbours.astype(jnp.int32).reshape(b * NEIGH), neigh_pad])

    T = _transform_table(features, W_dense, b_dense)
    pooled, nodef = _sc_gather_pool(T, features, neigh_flat, node_p)
    out = _final_matmul(nodef, pooled, neigh_weights[:D], neigh_weights[D:], b)
    return out


# speedup vs baseline: 3.6841x; 1.0196x over previous
import jax
import jax.numpy as jnp
from jax import lax
from jax.experimental import pallas as pl
from jax.experimental.pallas import tpu as pltpu
from jax.experimental.pallas import tpu_sc as plsc

D = 128
NEIGH = 32
NC = 2
NS = 16
B = 10000
QF = 320
QL = B - 31 * QF
C = 4
E = C * NEIGH
NBUF = 4
CN = 80


def _tc1_body(x_ref, w_ref, b_ref, o_ref):
    acc = jnp.dot(x_ref[...], w_ref[...], preferred_element_type=jnp.float32)
    o_ref[...] = jnp.maximum(acc + b_ref[...], 0.0)


def _transform_table(features, W_dense, b_dense):
    n = features.shape[0]
    blk = 1000
    grid = n // blk
    return pl.pallas_call(
        _tc1_body,
        grid=(grid,),
        in_specs=[
            pl.BlockSpec((blk, D), lambda i: (i, 0)),
            pl.BlockSpec((D, D), lambda i: (0, 0)),
            pl.BlockSpec((1, D), lambda i: (0, 0)),
        ],
        out_specs=pl.BlockSpec((blk, D), lambda i: (i, 0)),
        out_shape=jax.ShapeDtypeStruct((n, D), jnp.float32),
    )(features, W_dense, b_dense.reshape(1, D))


def _sc_body(t_hbm, feat_hbm, neigh_hbm, node_hbm, pooled_hbm, nodef_hbm,
             idxall_v, b0, b1, b2, b3, pooled_v, nidx_v,
             s0, s1, s2, s3):
    bufs = (b0, b1, b2, b3)
    sems = (s0, s1, s2, s3)
    cid = lax.axis_index("c")
    sid = lax.axis_index("s")
    wid = cid * NS + sid

    def gather_start(g, rows_v, sem):
        pltpu.async_copy(t_hbm.at[idxall_v.at[pl.ds(g * E, E)]], rows_v, sem)

    def gather_wait(rows_v, sem):
        pltpu.make_async_copy(t_hbm.at[idxall_v.at[pl.ds(0, E)]],
                              rows_v, sem).wait()

    def compute(g, rows_v):
        def nbody(n, accs):
            new = []
            for c in range(C):
                for d in range(D // 16):
                    new.append(accs[c * (D // 16) + d]
                               + rows_v[c * NEIGH + n, pl.ds(d * 16, 16)])
            return tuple(new)
        init = tuple(jnp.zeros((16,), jnp.float32)
                     for _ in range(C * (D // 16)))
        accs = lax.fori_loop(0, NEIGH, nbody, init)
        for c in range(C):
            for d in range(D // 16):
                pooled_v[g * C + c, pl.ds(d * 16, 16)] = (
                    accs[c * (D // 16) + d] * (1.0 / NEIGH))

    def run(base, rows):
        def node_chunk(j, carry):
            off = base + j * CN
            pltpu.sync_copy(node_hbm.at[pl.ds(off, CN)], nidx_v)
            pltpu.async_copy(feat_hbm.at[nidx_v],
                             pooled_v.at[pl.ds(0, CN)], s0).wait()
            pltpu.sync_copy(pooled_v.at[pl.ds(0, CN)],
                            nodef_hbm.at[pl.ds(off, CN)])
            return carry
        with jax.named_scope("node_gather"):
            lax.fori_loop(0, rows // CN, node_chunk, 0)

        with jax.named_scope("idx_prefetch"):
            pltpu.sync_copy(neigh_hbm.at[pl.ds(base * NEIGH, rows * NEIGH)],
                            idxall_v.at[pl.ds(0, rows * NEIGH)])

        nchunk = rows // C
        with jax.named_scope("pool_loop"):
            for b in range(NBUF - 1):
                gather_start(b, bufs[b], sems[b])

            def quad(i, carry):
                a = NBUF * i
                for b in range(NBUF):
                    g = a + b

                    @pl.when(g + NBUF - 1 < nchunk)
                    def _():
                        gather_start(g + NBUF - 1,
                                     bufs[(b + NBUF - 1) % NBUF],
                                     sems[(b + NBUF - 1) % NBUF])
                    gather_wait(bufs[b], sems[b])
                    compute(g, bufs[b])
                return carry
            lax.fori_loop(0, nchunk // NBUF, quad, 0)
        with jax.named_scope("pooled_writeout"):
            pltpu.sync_copy(pooled_v.at[pl.ds(0, rows)],
                            pooled_hbm.at[pl.ds(base, rows)])

    @pl.when(wid < 31)
    def _():
        run(wid * QF, QF)

    @pl.when(wid == 31)
    def _():
        run(31 * QF, QL)


def _sc_gather_pool(T, features, neigh_flat, node_flat):
    mesh = plsc.VectorSubcoreMesh(core_axis_name="c", subcore_axis_name="s")
    return pl.kernel(
        _sc_body,
        out_type=(
            jax.ShapeDtypeStruct((B, D), jnp.float32),
            jax.ShapeDtypeStruct((B, D), jnp.float32),
        ),
        mesh=mesh,
        scratch_types=[
            pltpu.VMEM((QF * NEIGH,), jnp.int32),
            pltpu.VMEM((E, D), jnp.float32),
            pltpu.VMEM((E, D), jnp.float32),
            pltpu.VMEM((E, D), jnp.float32),
            pltpu.VMEM((E, D), jnp.float32),
            pltpu.VMEM((QF, D), jnp.float32),
            pltpu.VMEM((CN,), jnp.int32),
            pltpu.SemaphoreType.DMA,
            pltpu.SemaphoreType.DMA,
            pltpu.SemaphoreType.DMA,
            pltpu.SemaphoreType.DMA,
        ],
    )(T, features, neigh_flat, node_flat)


def _tc2_body(nf_ref, pv_ref, w1_ref, w2_ref, o_ref):
    acc = jnp.dot(nf_ref[...], w1_ref[...], preferred_element_type=jnp.float32)
    acc = acc + jnp.dot(pv_ref[...], w2_ref[...], preferred_element_type=jnp.float32)
    o_ref[...] = jnp.maximum(acc, 0.0)


def _final_matmul(nodef, pooled, w1, w2, b):
    blk = 1000
    grid = b // blk
    return pl.pallas_call(
        _tc2_body,
        grid=(grid,),
        in_specs=[
            pl.BlockSpec((blk, D), lambda i: (i, 0)),
            pl.BlockSpec((blk, D), lambda i: (i, 0)),
            pl.BlockSpec((D, D), lambda i: (0, 0)),
            pl.BlockSpec((D, D), lambda i: (0, 0)),
        ],
        out_specs=pl.BlockSpec((blk, D), lambda i: (i, 0)),
        out_shape=jax.ShapeDtypeStruct((b, D), jnp.float32),
    )(nodef, pooled, w1, w2)


def kernel(features, node, neighbours, W_dense, b_dense, neigh_weights):
    b = node.shape[0]
    node_flat = node.reshape(b).astype(jnp.int32)
    neigh_flat = neighbours.astype(jnp.int32).reshape(b * NEIGH)

    T = _transform_table(features, W_dense, b_dense)
    pooled, nodef = _sc_gather_pool(T, features, neigh_flat, node_flat)
    out = _final_matmul(nodef, pooled, neigh_weights[:D], neigh_weights[D:], b)
    return out


# speedup vs baseline: 4.2944x; 1.1657x over previous
import jax
import jax.numpy as jnp
from jax import lax
from jax.experimental import pallas as pl
from jax.experimental.pallas import tpu as pltpu
from jax.experimental.pallas import tpu_sc as plsc

D = 128
NEIGH = 32
NC = 2
NS = 16
B = 10000
QF = 320
QL = B - 31 * QF
C = 4
E = C * NEIGH
NBUF = 4
CN = 80


def _tc1_body(x_ref, w_ref, b_ref, o_ref):
    acc = jnp.dot(x_ref[...], w_ref[...], preferred_element_type=jnp.float32)
    o_ref[...] = jnp.maximum(acc + b_ref[...], 0.0)


def _transform_table(features, W_dense, b_dense):
    n = features.shape[0]
    blk = 2000
    grid = n // blk
    return pl.pallas_call(
        _tc1_body,
        grid=(grid,),
        in_specs=[
            pl.BlockSpec((blk, D), lambda i: (i, 0)),
            pl.BlockSpec((D, D), lambda i: (0, 0)),
            pl.BlockSpec((1, D), lambda i: (0, 0)),
        ],
        out_specs=pl.BlockSpec((blk, D), lambda i: (i, 0)),
        out_shape=jax.ShapeDtypeStruct((n, D), jnp.float32),
    )(features, W_dense, b_dense.reshape(1, D))


def _sc_body(t_hbm, feat_hbm, neigh_hbm, node_hbm, pooled_hbm, nodef_hbm,
             idxall_v, b0, b1, b2, b3, pooled_v, nidx_v,
             s0, s1, s2, s3):
    bufs = (b0, b1, b2, b3)
    sems = (s0, s1, s2, s3)
    cid = lax.axis_index("c")
    sid = lax.axis_index("s")
    wid = cid * NS + sid

    def gather_start(g, rows_v, sem):
        pltpu.async_copy(t_hbm.at[idxall_v.at[pl.ds(g * E, E)]], rows_v, sem)

    def gather_wait(rows_v, sem):
        pltpu.make_async_copy(t_hbm.at[idxall_v.at[pl.ds(0, E)]],
                              rows_v, sem).wait()

    def compute(g, rows_v):
        def nbody(n, accs):
            new = []
            for c in range(C):
                for d in range(D // 16):
                    new.append(accs[c * (D // 16) + d]
                               + rows_v[c * NEIGH + n, pl.ds(d * 16, 16)])
            return tuple(new)
        init = tuple(jnp.zeros((16,), jnp.float32)
                     for _ in range(C * (D // 16)))
        accs = lax.fori_loop(0, NEIGH, nbody, init)
        for c in range(C):
            for d in range(D // 16):
                pooled_v[g * C + c, pl.ds(d * 16, 16)] = (
                    accs[c * (D // 16) + d] * (1.0 / NEIGH))

    def run(base, rows):
        def node_chunk(j, carry):
            off = base + j * CN
            pltpu.sync_copy(node_hbm.at[pl.ds(off, CN)], nidx_v)
            pltpu.async_copy(feat_hbm.at[nidx_v],
                             pooled_v.at[pl.ds(0, CN)], s0).wait()
            pltpu.sync_copy(pooled_v.at[pl.ds(0, CN)],
                            nodef_hbm.at[pl.ds(off, CN)])
            return carry
        with jax.named_scope("node_gather"):
            lax.fori_loop(0, rows // CN, node_chunk, 0)

        with jax.named_scope("idx_prefetch"):
            pltpu.sync_copy(neigh_hbm.at[pl.ds(base * NEIGH, rows * NEIGH)],
                            idxall_v.at[pl.ds(0, rows * NEIGH)])

        nchunk = rows // C
        with jax.named_scope("pool_loop"):
            for b in range(NBUF - 1):
                gather_start(b, bufs[b], sems[b])

            def quad(i, carry):
                a = NBUF * i
                for b in range(NBUF):
                    g = a + b

                    @pl.when(g + NBUF - 1 < nchunk)
                    def _():
                        gather_start(g + NBUF - 1,
                                     bufs[(b + NBUF - 1) % NBUF],
                                     sems[(b + NBUF - 1) % NBUF])
                    gather_wait(bufs[b], sems[b])
                    compute(g, bufs[b])
                return carry
            lax.fori_loop(0, nchunk // NBUF, quad, 0)
        with jax.named_scope("pooled_writeout"):
            pltpu.sync_copy(pooled_v.at[pl.ds(0, rows)],
                            pooled_hbm.at[pl.ds(base, rows)])

    @pl.when(wid < 31)
    def _():
        run(wid * QF, QF)

    @pl.when(wid == 31)
    def _():
        run(31 * QF, QL)


def _sc_gather_pool(T, features, neigh_flat, node_flat):
    mesh = plsc.VectorSubcoreMesh(core_axis_name="c", subcore_axis_name="s")
    return pl.kernel(
        _sc_body,
        out_type=(
            jax.ShapeDtypeStruct((B, D), jnp.float32),
            jax.ShapeDtypeStruct((B, D), jnp.float32),
        ),
        mesh=mesh,
        scratch_types=[
            pltpu.VMEM((QF * NEIGH,), jnp.int32),
            pltpu.VMEM((E, D), jnp.float32),
            pltpu.VMEM((E, D), jnp.float32),
            pltpu.VMEM((E, D), jnp.float32),
            pltpu.VMEM((E, D), jnp.float32),
            pltpu.VMEM((QF, D), jnp.float32),
            pltpu.VMEM((CN,), jnp.int32),
            pltpu.SemaphoreType.DMA,
            pltpu.SemaphoreType.DMA,
            pltpu.SemaphoreType.DMA,
            pltpu.SemaphoreType.DMA,
        ],
    )(T, features, neigh_flat, node_flat)


def _tc2_body(nf_ref, pv_ref, w1_ref, w2_ref, o_ref):
    acc = jnp.dot(nf_ref[...], w1_ref[...], preferred_element_type=jnp.float32)
    acc = acc + jnp.dot(pv_ref[...], w2_ref[...], preferred_element_type=jnp.float32)
    o_ref[...] = jnp.maximum(acc, 0.0)


def _final_matmul(nodef, pooled, w1, w2, b):
    blk = 1000
    grid = b // blk
    return pl.pallas_call(
        _tc2_body,
        grid=(grid,),
        in_specs=[
            pl.BlockSpec((blk, D), lambda i: (i, 0)),
            pl.BlockSpec((blk, D), lambda i: (i, 0)),
            pl.BlockSpec((D, D), lambda i: (0, 0)),
            pl.BlockSpec((D, D), lambda i: (0, 0)),
        ],
        out_specs=pl.BlockSpec((blk, D), lambda i: (i, 0)),
        out_shape=jax.ShapeDtypeStruct((b, D), jnp.float32),
    )(nodef, pooled, w1, w2)


def kernel(features, node, neighbours, W_dense, b_dense, neigh_weights):
    b = node.shape[0]
    node_flat = node.reshape(b).astype(jnp.int32)
    neigh_flat = neighbours.astype(jnp.int32).reshape(b * NEIGH)

    T = _transform_table(features, W_dense, b_dense)
    pooled, nodef = _sc_gather_pool(T, features, neigh_flat, node_flat)
    out = _final_matmul(nodef, pooled, neigh_weights[:D], neigh_weights[D:], b)
    return out


# speedup vs baseline: 4.8232x; 1.1232x over previous
import jax
import jax.numpy as jnp
from jax import lax
from jax.experimental import pallas as pl
from jax.experimental.pallas import tpu as pltpu
from jax.experimental.pallas import tpu_sc as plsc

D = 128
NEIGH = 32
NC = 2
NS = 16
B = 10000
QF = 320
QL = B - 31 * QF
C = 4
E = C * NEIGH
NBUF = 4
CN = 80


def _tc1_body(x_ref, w_ref, b_ref, o_ref):
    acc = jnp.dot(x_ref[...], w_ref[...], preferred_element_type=jnp.float32)
    o_ref[...] = jnp.maximum(acc + b_ref[...], 0.0)


def _transform_table(features, W_dense, b_dense):
    n = features.shape[0]
    blk = 5000
    grid = n // blk
    return pl.pallas_call(
        _tc1_body,
        grid=(grid,),
        in_specs=[
            pl.BlockSpec((blk, D), lambda i: (i, 0)),
            pl.BlockSpec((D, D), lambda i: (0, 0)),
            pl.BlockSpec((1, D), lambda i: (0, 0)),
        ],
        out_specs=pl.BlockSpec((blk, D), lambda i: (i, 0)),
        out_shape=jax.ShapeDtypeStruct((n, D), jnp.float32),
    )(features, W_dense, b_dense.reshape(1, D))


def _sc_body(t_hbm, feat_hbm, neigh_hbm, node_hbm, pooled_hbm, nodef_hbm,
             idxall_v, b0, b1, b2, b3, pooled_v, nidx_v,
             s0, s1, s2, s3):
    bufs = (b0, b1, b2, b3)
    sems = (s0, s1, s2, s3)
    cid = lax.axis_index("c")
    sid = lax.axis_index("s")
    wid = cid * NS + sid

    def gather_start(g, rows_v, sem):
        pltpu.async_copy(t_hbm.at[idxall_v.at[pl.ds(g * E, E)]], rows_v, sem)

    def gather_wait(rows_v, sem):
        pltpu.make_async_copy(t_hbm.at[idxall_v.at[pl.ds(0, E)]],
                              rows_v, sem).wait()

    def compute(g, rows_v):
        def nbody(n, accs):
            new = []
            for c in range(C):
                for d in range(D // 16):
                    new.append(accs[c * (D // 16) + d]
                               + rows_v[c * NEIGH + n, pl.ds(d * 16, 16)])
            return tuple(new)
        init = tuple(jnp.zeros((16,), jnp.float32)
                     for _ in range(C * (D // 16)))
        accs = lax.fori_loop(0, NEIGH, nbody, init)
        for c in range(C):
            for d in range(D // 16):
                pooled_v[g * C + c, pl.ds(d * 16, 16)] = (
                    accs[c * (D // 16) + d] * (1.0 / NEIGH))

    def run(base, rows):
        def node_chunk(j, carry):
            off = base + j * CN
            pltpu.sync_copy(node_hbm.at[pl.ds(off, CN)], nidx_v)
            pltpu.async_copy(feat_hbm.at[nidx_v],
                             pooled_v.at[pl.ds(0, CN)], s0).wait()
            pltpu.sync_copy(pooled_v.at[pl.ds(0, CN)],
                            nodef_hbm.at[pl.ds(off, CN)])
            return carry
        with jax.named_scope("node_gather"):
            lax.fori_loop(0, rows // CN, node_chunk, 0)

        with jax.named_scope("idx_prefetch"):
            pltpu.sync_copy(neigh_hbm.at[pl.ds(base * NEIGH, rows * NEIGH)],
                            idxall_v.at[pl.ds(0, rows * NEIGH)])

        nchunk = rows // C
        with jax.named_scope("pool_loop"):
            for b in range(NBUF - 1):
                gather_start(b, bufs[b], sems[b])

            def quad(i, carry):
                a = NBUF * i
                for b in range(NBUF):
                    g = a + b

                    @pl.when(g + NBUF - 1 < nchunk)
                    def _():
                        gather_start(g + NBUF - 1,
                                     bufs[(b + NBUF - 1) % NBUF],
                                     sems[(b + NBUF - 1) % NBUF])
                    gather_wait(bufs[b], sems[b])
                    compute(g, bufs[b])
                return carry
            lax.fori_loop(0, nchunk // NBUF, quad, 0)
        with jax.named_scope("pooled_writeout"):
            pltpu.sync_copy(pooled_v.at[pl.ds(0, rows)],
                            pooled_hbm.at[pl.ds(base, rows)])

    @pl.when(wid < 31)
    def _():
        run(wid * QF, QF)

    @pl.when(wid == 31)
    def _():
        run(31 * QF, QL)


def _sc_gather_pool(T, features, neigh_flat, node_flat):
    mesh = plsc.VectorSubcoreMesh(core_axis_name="c", subcore_axis_name="s")
    return pl.kernel(
        _sc_body,
        out_type=(
            jax.ShapeDtypeStruct((B, D), jnp.float32),
            jax.ShapeDtypeStruct((B, D), jnp.float32),
        ),
        mesh=mesh,
        scratch_types=[
            pltpu.VMEM((QF * NEIGH,), jnp.int32),
            pltpu.VMEM((E, D), jnp.float32),
            pltpu.VMEM((E, D), jnp.float32),
            pltpu.VMEM((E, D), jnp.float32),
            pltpu.VMEM((E, D), jnp.float32),
            pltpu.VMEM((QF, D), jnp.float32),
            pltpu.VMEM((CN,), jnp.int32),
            pltpu.SemaphoreType.DMA,
            pltpu.SemaphoreType.DMA,
            pltpu.SemaphoreType.DMA,
            pltpu.SemaphoreType.DMA,
        ],
    )(T, features, neigh_flat, node_flat)


def _tc2_body(nf_ref, pv_ref, w1_ref, w2_ref, o_ref):
    acc = jnp.dot(nf_ref[...], w1_ref[...], preferred_element_type=jnp.float32)
    acc = acc + jnp.dot(pv_ref[...], w2_ref[...], preferred_element_type=jnp.float32)
    o_ref[...] = jnp.maximum(acc, 0.0)


def _final_matmul(nodef, pooled, w1, w2, b):
    blk = 1000
    grid = b // blk
    return pl.pallas_call(
        _tc2_body,
        grid=(grid,),
        in_specs=[
            pl.BlockSpec((blk, D), lambda i: (i, 0)),
            pl.BlockSpec((blk, D), lambda i: (i, 0)),
            pl.BlockSpec((D, D), lambda i: (0, 0)),
            pl.BlockSpec((D, D), lambda i: (0, 0)),
        ],
        out_specs=pl.BlockSpec((blk, D), lambda i: (i, 0)),
        out_shape=jax.ShapeDtypeStruct((b, D), jnp.float32),
    )(nodef, pooled, w1, w2)


def kernel(features, node, neighbours, W_dense, b_dense, neigh_weights):
    b = node.shape[0]
    node_flat = node.reshape(b).astype(jnp.int32)
    neigh_flat = neighbours.astype(jnp.int32).reshape(b * NEIGH)

    T = _transform_table(features, W_dense, b_dense)
    pooled, nodef = _sc_gather_pool(T, features, neigh_flat, node_flat)
    out = _final_matmul(nodef, pooled, neigh_weights[:D], neigh_weights[D:], b)
    return out
